# baseline (speedup 1.0000x reference)
import functools

import jax
import jax.numpy as jnp
from jax.experimental import pallas as pl
from jax.experimental.pallas import tpu as pltpu

EPS_LN = 1e-6


def _knn_interp_body(npow2_bits, ca_ref, xt_ref, f_ref, g_ref, b_ref, o_ref):
    n = ca_ref.shape[2]
    tm = xt_ref.shape[1]
    d2f = jax.lax.dot_general(
        xt_ref[0], ca_ref[0],
        dimension_numbers=(((1,), (0,)), ((), ())),
        precision=jax.lax.Precision.HIGHEST,
        preferred_element_type=jnp.float32)
    big = jnp.int32(2**31 - 1)
    nb = 128
    m1 = jnp.full((tm, nb), big, jnp.int32)
    m2 = m1
    m3 = m1
    for g in range(n // nb):
        lane_c = jax.lax.broadcasted_iota(jnp.int32, (tm, nb), 1) + g * nb
        kc = (d2f[:, g * nb:(g + 1) * nb].astype(jnp.int32)
              << npow2_bits) | lane_c
        lo = jnp.minimum(m1, kc)
        hi = jnp.maximum(m1, kc)
        m1 = lo
        lo = jnp.minimum(m2, hi)
        hi = jnp.maximum(m2, hi)
        m2 = lo
        m3 = jnp.minimum(m3, hi)
    ks = jnp.concatenate([m1, m2, m3], axis=1)
    k1 = jnp.min(ks, axis=1, keepdims=True)
    ks = jnp.where(ks == k1, big, ks)
    k2 = jnp.min(ks, axis=1, keepdims=True)
    ks = jnp.where(ks == k2, big, ks)
    k3 = jnp.min(ks, axis=1, keepdims=True)

    lane = jax.lax.broadcasted_iota(jnp.int32, (tm, n), 1)
    mask = jnp.int32((1 << npow2_bits) - 1)
    w_list = []
    idx_list = []
    for k in (k1, k2, k3):
        idx_list.append(k & mask)
        d2f = (k >> npow2_bits).astype(jnp.float32)
        w_list.append(1.0 / (d2f + 1e-8))
    norm = w_list[0] + w_list[1] + w_list[2]
    w_list = [w / norm for w in w_list]

    zero = jnp.float32(0.0)
    W = jnp.where(lane == idx_list[0], w_list[0], zero)
    W = jnp.where(lane == idx_list[1], w_list[1], W)
    W = jnp.where(lane == idx_list[2], w_list[2], W)
    interp = jnp.dot(W, f_ref[0], preferred_element_type=jnp.float32)

    mu = jnp.mean(interp, axis=1, keepdims=True)
    xc = interp - mu
    var = jnp.mean(xc * xc, axis=1, keepdims=True)
    y = xc / jnp.sqrt(var + EPS_LN) * g_ref[0:1, :] + b_ref[0:1, :]
    o_ref[0] = y * 0.5 * (1.0 + jax.lax.erf(y * 0.7071067811865476))


def _sort_keys(c):
    step = c.max() + 1
    return c[..., 0] + c[..., 1] * step + c[..., 2] * step * step


def kernel(features, coords, xyz_t, gamma, beta):
    b, n, c = features.shape
    m = xyz_t.shape[1]
    npow2_bits = max(1, (n - 1).bit_length())

    order = jnp.argsort(_sort_keys(coords), axis=1)
    order_t = jnp.argsort(_sort_keys(xyz_t), axis=1)
    c_sorted = jnp.take_along_axis(coords, order[..., None], axis=1)
    xt_s = jnp.take_along_axis(xyz_t, order_t[..., None], axis=1)
    f_sorted = jnp.take_along_axis(features, order[..., None], axis=1)

    c_f = c_sorted.astype(jnp.float32)
    xt_ff = xt_s.astype(jnp.float32)
    ca_pad = (jnp.zeros((b, 8, n), jnp.float32)
              .at[:, 0:3, :].set(-2.0 * c_f.transpose(0, 2, 1))
              .at[:, 3, :].set(jnp.sum(c_f * c_f, axis=-1))
              .at[:, 4, :].set(1.0))
    xt_pad = (jnp.zeros((b, m, 8), jnp.float32)
              .at[:, :, 0:3].set(xt_ff)
              .at[:, :, 3].set(1.0)
              .at[:, :, 4].set(jnp.sum(xt_ff * xt_ff, axis=-1)))

    tm = 256
    grid = (b, m // tm)
    out = pl.pallas_call(
        functools.partial(_knn_interp_body, npow2_bits),
        grid=grid,
        in_specs=[
            pl.BlockSpec((1, 8, n), lambda i, j: (i, 0, 0)),
            pl.BlockSpec((1, tm, 8), lambda i, j: (i, j, 0)),
            pl.BlockSpec((1, n, c), lambda i, j: (i, 0, 0)),
            pl.BlockSpec((1, c), lambda i, j: (0, 0)),
            pl.BlockSpec((1, c), lambda i, j: (0, 0)),
        ],
        out_specs=pl.BlockSpec((1, tm, c), lambda i, j: (i, j, 0)),
        out_shape=jax.ShapeDtypeStruct((b, m, c), jnp.float32),
    )(ca_pad, xt_pad, f_sorted, gamma.reshape(1, c), beta.reshape(1, c))

    out_feats = out.reshape(b * m, c)
    xt_f = xt_s.reshape(b * m, 3).astype(jnp.float32)
    bcol = jnp.repeat(jnp.arange(b, dtype=jnp.float32), m)[:, None]
    out_coords = jnp.concatenate([bcol, xt_f], axis=1)
    return out_feats, out_coords


# speedup vs baseline: 1.7015x; 1.7015x over previous
import functools

import jax
import jax.numpy as jnp
from jax.experimental import pallas as pl
from jax.experimental.pallas import tpu as pltpu

EPS_LN = 1e-6


def _knn_interp_body(npow2_bits, ct_ref, xt_ref, f_ref, g_ref, b_ref, o_ref):
    n = ct_ref.shape[2]
    tm = xt_ref.shape[1]
    cx = ct_ref[0, 0:1, :]
    cy = ct_ref[0, 1:2, :]
    cz = ct_ref[0, 2:3, :]
    tx = xt_ref[0, :, 0:1]
    ty = xt_ref[0, :, 1:2]
    tz = xt_ref[0, :, 2:3]
    dx = tx - cx
    dy = ty - cy
    dz = tz - cz
    d2 = dx * dx + dy * dy + dz * dz
    nb = 128
    bigf = jnp.float32(3e9)
    m1 = jnp.full((tm, nb), bigf, jnp.float32)
    m2 = m1
    m3 = m1
    flag = 1 << 30
    for g in range(n // nb):
        lane_c = (jax.lax.broadcasted_iota(jnp.int32, (tm, nb), 1)
                  + (g * nb | flag))
        kc_i = (d2[:, g * nb:(g + 1) * nb] << npow2_bits) | lane_c
        kc = jax.lax.bitcast_convert_type(kc_i, jnp.float32)
        lo = jnp.minimum(m1, kc)
        hi = jnp.maximum(m1, kc)
        m1 = lo
        lo = jnp.minimum(m2, hi)
        hi = jnp.maximum(m2, hi)
        m2 = lo
        m3 = jnp.minimum(m3, hi)
    ks = jnp.concatenate([m1, m2, m3], axis=1)
    k1 = jnp.min(ks, axis=1, keepdims=True)
    ks = jnp.where(ks == k1, bigf, ks)
    k2 = jnp.min(ks, axis=1, keepdims=True)
    ks = jnp.where(ks == k2, bigf, ks)
    k3 = jnp.min(ks, axis=1, keepdims=True)

    lane = jax.lax.broadcasted_iota(jnp.int32, (tm, n), 1)
    mask = jnp.int32((1 << npow2_bits) - 1)
    w_list = []
    idx_list = []
    for k in (k1, k2, k3):
        ki = jax.lax.bitcast_convert_type(k, jnp.int32) & (flag - 1)
        idx_list.append(ki & mask)
        d2f = (ki >> npow2_bits).astype(jnp.float32)
        w_list.append(1.0 / (d2f + 1e-8))
    norm = w_list[0] + w_list[1] + w_list[2]
    w_list = [w / norm for w in w_list]

    zero = jnp.float32(0.0)
    W = jnp.where(lane == idx_list[0], w_list[0], zero)
    W = jnp.where(lane == idx_list[1], w_list[1], W)
    W = jnp.where(lane == idx_list[2], w_list[2], W)
    interp = jnp.dot(W, f_ref[0], preferred_element_type=jnp.float32)

    mu = jnp.mean(interp, axis=1, keepdims=True)
    xc = interp - mu
    var = jnp.mean(xc * xc, axis=1, keepdims=True)
    y = xc / jnp.sqrt(var + EPS_LN) * g_ref[0:1, :] + b_ref[0:1, :]
    o_ref[0] = y * 0.5 * (1.0 + jax.lax.erf(y * 0.7071067811865476))


def _sort_keys(c):
    step = c.max() + 1
    return c[..., 0] + c[..., 1] * step + c[..., 2] * step * step


def kernel(features, coords, xyz_t, gamma, beta):
    b, n, c = features.shape
    m = xyz_t.shape[1]
    npow2_bits = max(1, (n - 1).bit_length())

    order = jnp.argsort(_sort_keys(coords), axis=1)
    order_t = jnp.argsort(_sort_keys(xyz_t), axis=1)
    c_sorted = jnp.take_along_axis(coords, order[..., None], axis=1)
    xt_s = jnp.take_along_axis(xyz_t, order_t[..., None], axis=1)
    f_sorted = jnp.take_along_axis(features, order[..., None], axis=1)

    ct_pad = jnp.zeros((b, 8, n), jnp.int32).at[:, 0:3, :].set(
        c_sorted.transpose(0, 2, 1))
    xt_pad = jnp.zeros((b, m, 8), jnp.int32).at[:, :, 0:3].set(xt_s)

    tm = 256
    grid = (b, m // tm)
    out = pl.pallas_call(
        functools.partial(_knn_interp_body, npow2_bits),
        grid=grid,
        in_specs=[
            pl.BlockSpec((1, 8, n), lambda i, j: (i, 0, 0)),
            pl.BlockSpec((1, tm, 8), lambda i, j: (i, j, 0)),
            pl.BlockSpec((1, n, c), lambda i, j: (i, 0, 0)),
            pl.BlockSpec((1, c), lambda i, j: (0, 0)),
            pl.BlockSpec((1, c), lambda i, j: (0, 0)),
        ],
        out_specs=pl.BlockSpec((1, tm, c), lambda i, j: (i, j, 0)),
        out_shape=jax.ShapeDtypeStruct((b, m, c), jnp.float32),
    )(ct_pad, xt_pad, f_sorted, gamma.reshape(1, c), beta.reshape(1, c))

    out_feats = out.reshape(b * m, c)
    xt_f = xt_s.reshape(b * m, 3).astype(jnp.float32)
    bcol = jnp.repeat(jnp.arange(b, dtype=jnp.float32), m)[:, None]
    out_coords = jnp.concatenate([bcol, xt_f], axis=1)
    return out_feats, out_coords
